# baseline (speedup 1.0000x reference)
import functools
import jax
import jax.numpy as jnp
from jax import lax
from jax.experimental import pallas as pl
from jax.experimental.pallas import tpu as pltpu
from jax.experimental.pallas import tpu_sc as plsc

_BATCH = 16384
_DIM = 128
_TABLE_ROWS = 6
_EPS = 1e-6

_NW = 32
_ROWS_PER_W = _BATCH // _NW
_CHUNK = 128
_N_CHUNKS = _ROWS_PER_W // _CHUNK
_GROUPS = _CHUNK // 16


def _rsqrt16(v):
    i = lax.bitcast_convert_type(v, jnp.int32)
    i = jnp.int32(0x5F3759DF) - lax.shift_right_logical(i, 1)
    y = lax.bitcast_convert_type(i, jnp.float32)
    h = v * 0.5
    y = y * (1.5 - h * y * y)
    y = y * (1.5 - h * y * y)
    y = y * (1.5 - h * y * y)
    return y


def _sc_body(x_hbm, gamma_hbm, beta_hbm, table_hbm, dom_hbm, out_hbm,
             x_v, gamma_v, beta_v, table_v, dom_v):
    wid = lax.axis_index("s") * 2 + lax.axis_index("c")
    base = wid * _ROWS_PER_W

    pltpu.sync_copy(gamma_hbm, gamma_v)
    pltpu.sync_copy(beta_hbm, beta_v)
    pltpu.sync_copy(table_hbm, table_v)
    pltpu.sync_copy(dom_hbm, dom_v)
    d = dom_v[...][0]

    zeros16 = jnp.zeros((16,), jnp.float32)
    gam = []
    cbv = []
    for jj in range(_DIM // 16):
        s = pl.ds(jj * 16, 16)
        gam.append(gamma_v[s])
        c = beta_v[s]
        for r in range(_TABLE_ROWS):
            c = c + jnp.where(d == r, table_v[r, s], zeros16)
        cbv.append(c)

    iota16 = lax.iota(jnp.int32, 16)
    inv_dim = jnp.float32(1.0 / _DIM)

    for chunk in range(_N_CHUNKS):
        row_base = base + chunk * _CHUNK
        pltpu.sync_copy(x_hbm.at[pl.ds(row_base, _CHUNK)], x_v)

        def group_body(g, _):
            r0 = g * 16
            idx_r = iota16 + r0
            acc = zeros16
            acc2 = zeros16
            for j in range(_DIM):
                idx_c = jnp.full((16,), j, jnp.int32)
                x = plsc.load_gather(x_v, [idx_r, idx_c])
                acc = acc + x
                acc2 = acc2 + x * x
            mean = acc * inv_dim
            var = acc2 * inv_dim - mean * mean
            inv = _rsqrt16(var + _EPS)
            for rr in range(16):
                m = mean[rr]
                iv = inv[rr]
                for jj in range(_DIM // 16):
                    s = pl.ds(jj * 16, 16)
                    x_v[r0 + rr, s] = ((x_v[r0 + rr, s] - m) * iv) * gam[jj] + cbv[jj]
            return 0

        lax.fori_loop(0, _GROUPS, group_body, 0)
        pltpu.sync_copy(x_v, out_hbm.at[pl.ds(row_base, _CHUNK)])


def kernel(inputs, gamma, beta, inner_bias, domain):
    dom16 = jnp.full((16,), jnp.asarray(domain, jnp.int32))
    mesh = plsc.VectorSubcoreMesh(core_axis_name="c", subcore_axis_name="s")
    k = pl.kernel(
        _sc_body,
        mesh=mesh,
        out_type=jax.ShapeDtypeStruct((_BATCH, _DIM), jnp.float32),
        compiler_params=pltpu.CompilerParams(needs_layout_passes=False),
        scratch_types=[
            pltpu.VMEM((_CHUNK, _DIM), jnp.float32),
            pltpu.VMEM((_DIM,), jnp.float32),
            pltpu.VMEM((_DIM,), jnp.float32),
            pltpu.VMEM((_TABLE_ROWS, _DIM), jnp.float32),
            pltpu.VMEM((16,), jnp.int32),
        ],
    )
    return k(inputs, gamma, beta, inner_bias, dom16)


# speedup vs baseline: 1.0967x; 1.0967x over previous
import jax
import jax.numpy as jnp
from jax import lax
from jax.experimental import pallas as pl
from jax.experimental.pallas import tpu as pltpu
from jax.experimental.pallas import tpu_sc as plsc

_BATCH = 16384
_DIM = 128
_TABLE_ROWS = 6
_EPS = 1e-6

_NW = 32
_ROWS_PER_W = _BATCH // _NW
_CHUNK = 128
_N_CHUNKS = _ROWS_PER_W // _CHUNK
_GROUPS = _CHUNK // 16


def _rsqrt16(v):
    i = lax.bitcast_convert_type(v, jnp.int32)
    i = jnp.int32(0x5F3759DF) - lax.shift_right_logical(i, 1)
    y = lax.bitcast_convert_type(i, jnp.float32)
    h = v * 0.5
    y = y * (1.5 - h * y * y)
    y = y * (1.5 - h * y * y)
    y = y * (1.5 - h * y * y)
    return y


def _sc_body(x_hbm, gamma_hbm, beta_hbm, table_hbm, dom_hbm, out_hbm,
             x_v0, x_v1, x_v2, x_v3, gamma_v, beta_v, table_v, dom_v,
             in_sem, out_sem):
    bufs = (x_v0, x_v1, x_v2, x_v3)
    wid = lax.axis_index("s") * 2 + lax.axis_index("c")
    base = wid * _ROWS_PER_W

    in_copies = []
    for c in range(_N_CHUNKS):
        cp = pltpu.make_async_copy(
            x_hbm.at[pl.ds(base + c * _CHUNK, _CHUNK)], bufs[c], in_sem)
        cp.start()
        in_copies.append(cp)

    pltpu.sync_copy(gamma_hbm, gamma_v)
    pltpu.sync_copy(beta_hbm, beta_v)
    pltpu.sync_copy(table_hbm, table_v)
    pltpu.sync_copy(dom_hbm, dom_v)
    d = dom_v[...][0]

    zeros16 = jnp.zeros((16,), jnp.float32)
    gam = []
    cbv = []
    for jj in range(_DIM // 16):
        s = pl.ds(jj * 16, 16)
        gam.append(gamma_v[s])
        c = beta_v[s]
        for r in range(_TABLE_ROWS):
            c = c + jnp.where(d == r, table_v[r, s], zeros16)
        cbv.append(c)

    iota16 = lax.iota(jnp.int32, 16)
    inv_dim = jnp.float32(1.0 / _DIM)
    out_copies = []

    for c in range(_N_CHUNKS):
        x_v = bufs[c]
        in_copies[c].wait()

        def group_body(g, _, x_v=x_v):
            r0 = g * 16
            idx_r = iota16 + r0
            acc = zeros16
            acc2 = zeros16
            for j in range(_DIM):
                idx_c = jnp.full((16,), j, jnp.int32)
                x = plsc.load_gather(x_v, [idx_r, idx_c])
                acc = acc + x
                acc2 = acc2 + x * x
            mean = acc * inv_dim
            var = acc2 * inv_dim - mean * mean
            inv = _rsqrt16(var + _EPS)
            for rr in range(16):
                m = mean[rr]
                iv = inv[rr]
                for jj in range(_DIM // 16):
                    s = pl.ds(jj * 16, 16)
                    x_v[r0 + rr, s] = ((x_v[r0 + rr, s] - m) * iv) * gam[jj] + cbv[jj]
            return 0

        lax.fori_loop(0, _GROUPS, group_body, 0)
        cp = pltpu.make_async_copy(
            x_v, out_hbm.at[pl.ds(base + c * _CHUNK, _CHUNK)], out_sem)
        cp.start()
        out_copies.append(cp)

    for cp in out_copies:
        cp.wait()


def kernel(inputs, gamma, beta, inner_bias, domain):
    dom16 = jnp.full((16,), jnp.asarray(domain, jnp.int32))
    mesh = plsc.VectorSubcoreMesh(core_axis_name="c", subcore_axis_name="s")
    k = pl.kernel(
        _sc_body,
        mesh=mesh,
        out_type=jax.ShapeDtypeStruct((_BATCH, _DIM), jnp.float32),
        compiler_params=pltpu.CompilerParams(needs_layout_passes=False),
        scratch_types=[
            pltpu.VMEM((_CHUNK, _DIM), jnp.float32),
            pltpu.VMEM((_CHUNK, _DIM), jnp.float32),
            pltpu.VMEM((_CHUNK, _DIM), jnp.float32),
            pltpu.VMEM((_CHUNK, _DIM), jnp.float32),
            pltpu.VMEM((_DIM,), jnp.float32),
            pltpu.VMEM((_DIM,), jnp.float32),
            pltpu.VMEM((_TABLE_ROWS, _DIM), jnp.float32),
            pltpu.VMEM((16,), jnp.int32),
            pltpu.SemaphoreType.DMA,
            pltpu.SemaphoreType.DMA,
        ],
    )
    return k(inputs, gamma, beta, inner_bias, dom16)


# speedup vs baseline: 7.3121x; 6.6673x over previous
import jax
import jax.numpy as jnp
from jax.experimental import pallas as pl
from jax.experimental.pallas import tpu as pltpu

_BATCH = 16384
_DIM = 128
_TABLE_ROWS = 6
_EPS = 1e-6
_TILES = (512, 512, 1024, 2048, 2048, 2048, 2048, 2048, 2048, 1024, 512, 512)
_OFFS = tuple(sum(_TILES[:i]) for i in range(len(_TILES)))


def _ln_block(x, gamma, bias):
    mean = jnp.mean(x, axis=1, keepdims=True)
    xc = x - mean
    var = jnp.mean(xc * xc, axis=1, keepdims=True)
    inv = jax.lax.rsqrt(var + _EPS)
    return xc * inv * gamma + bias


def _mb_kernel(dom_ref, x_hbm, gamma_ref, beta_ref, table_ref, o_hbm, *scr):
    n = len(_TILES)
    xbufs, obufs = scr[:n], scr[n:2 * n]
    insems, outsems = scr[2 * n], scr[2 * n + 1]
    d = dom_ref[0]
    gamma = gamma_ref[...]
    bias = beta_ref[...] + table_ref[pl.ds(d, 1), :]

    def in_copy(t):
        return pltpu.make_async_copy(
            x_hbm.at[pl.ds(_OFFS[t], _TILES[t])], xbufs[t], insems.at[t])

    def out_copy(t):
        return pltpu.make_async_copy(
            obufs[t], o_hbm.at[pl.ds(_OFFS[t], _TILES[t])], outsems.at[t])

    for t in range(n):
        in_copy(t).start()
    for t in range(n):
        in_copy(t).wait()
        obufs[t][...] = _ln_block(xbufs[t][...], gamma, bias)
        out_copy(t).start()
    for t in range(n):
        out_copy(t).wait()


def kernel(inputs, gamma, beta, inner_bias, domain):
    dom = jnp.asarray(domain, dtype=jnp.int32).reshape((1,))
    gamma2 = gamma.reshape(1, _DIM)
    beta2 = beta.reshape(1, _DIM)
    return pl.pallas_call(
        _mb_kernel,
        in_specs=[
            pl.BlockSpec(memory_space=pltpu.SMEM),
            pl.BlockSpec(memory_space=pl.ANY),
            pl.BlockSpec((1, _DIM), lambda: (0, 0)),
            pl.BlockSpec((1, _DIM), lambda: (0, 0)),
            pl.BlockSpec((_TABLE_ROWS, _DIM), lambda: (0, 0)),
        ],
        out_specs=pl.BlockSpec(memory_space=pl.ANY),
        out_shape=jax.ShapeDtypeStruct((_BATCH, _DIM), jnp.float32),
        scratch_shapes=(
            [pltpu.VMEM((t, _DIM), jnp.float32) for t in _TILES]
            + [pltpu.VMEM((t, _DIM), jnp.float32) for t in _TILES]
            + [pltpu.SemaphoreType.DMA((len(_TILES),)),
               pltpu.SemaphoreType.DMA((len(_TILES),))]
        ),
    )(dom, inputs, gamma2, beta2, inner_bias)
